# baseline (speedup 1.0000x reference)
import numpy as np
import jax
import jax.numpy as jnp
from jax import lax
from jax.experimental import pallas as pl
from jax.experimental.pallas import tpu as pltpu
from jax.experimental.pallas import tpu_sc as plsc

N_NEG = 24
TEMP = 0.07
B, D, H, W = 16, 32, 224, 224
TB = 2 * B
HW = H * W

_perms = np.stack([np.random.default_rng(1000 + i).permutation(TB)[:N_NEG]
                   for i in range(TB)])
_sel = np.zeros((TB, TB), np.float32)
for _i in range(TB):
    _sel[_i, _perms[_i]] = 1.0


def _logits_kernel(feat_ref, lab_ref, out_ref, acc_ref):
    i = pl.program_id(0)
    f = feat_ref[0]
    l0 = lab_ref[0, 0]
    l1 = lab_ref[0, 1]
    st0 = jnp.sum(f * l0[None, :, :], axis=(1, 2), keepdims=True)
    st1 = jnp.sum(f * l1[None, :, :], axis=(1, 2), keepdims=True)
    tot = jnp.sum(f, axis=(1, 2), keepdims=True)
    acc_ref[i, :, 0:1] = st0.reshape(D, 1)
    acc_ref[i, :, 1:2] = st1.reshape(D, 1)
    acc_ref[i, :, 2:3] = tot.reshape(D, 1)
    cnt0 = jnp.sum(l0, axis=(0, 1), keepdims=True)
    cnt1 = jnp.sum(l1, axis=(0, 1), keepdims=True)
    acc_ref[i, 0:1, 3:4] = cnt0
    acc_ref[i, 0:1, 4:5] = cnt1

    @pl.when(i == B - 1)
    def _finish():
        st_t = jnp.concatenate([acc_ref[b, :, 0:2] for b in range(B)], axis=1)
        tot_t = jnp.concatenate(
            [acc_ref[b, :, 2:3] for b in range(B) for _ in range(2)], axis=1)
        cnt_r = jnp.concatenate(
            [acc_ref[b, 0:1, 3:5] for b in range(B)], axis=1)

        def normalize(v, c):
            v = v / jnp.maximum(c, 1.0)
            n = jnp.sqrt(jnp.sum(v * v, axis=0, keepdims=True))
            return v / jnp.maximum(n, 1e-12)

        tgt = normalize(st_t, cnt_r)
        bgd = normalize(tot_t - st_t, float(HW) - cnt_r)
        dn = (((0,), (0,)), ((), ()))
        out_ref[0] = jax.lax.dot_general(
            tgt, tgt, dn, preferred_element_type=jnp.float32) / TEMP
        out_ref[1] = jax.lax.dot_general(
            tgt, bgd, dn, preferred_element_type=jnp.float32) / TEMP


def _sc_select_kernel(lt_hbm, lb_hbm, tid_hbm, sel_hbm, out_hbm,
                      ltrow, lbrow, selrow, tidrow, obuf):
    nc = 2
    wid = lax.axis_index("s") * nc + lax.axis_index("c")
    pltpu.sync_copy(lt_hbm.at[wid], ltrow)
    pltpu.sync_copy(lb_hbm.at[wid], lbrow)
    pltpu.sync_copy(sel_hbm.at[wid], selrow)
    pltpu.sync_copy(tid_hbm, tidrow)

    half = TB // 2
    ci_a = lax.iota(jnp.int32, 16)
    ci_b = ci_a + half
    tid_a = tidrow[pl.ds(0, half)]
    tid_b = tidrow[pl.ds(half, half)]
    tid_i = (jnp.sum(jnp.where(ci_a == wid, tid_a, 0))
             + jnp.sum(jnp.where(ci_b == wid, tid_b, 0)))

    same_a = tid_a == tid_i
    same_b = tid_b == tid_i
    df_a = jnp.where(same_a, 0, 1)
    df_b = jnp.where(same_b, 0, 1)
    sm_a = 1 - df_a
    sm_b = 1 - df_b
    rd_a = jnp.cumsum(df_a) - df_a
    rd_b = jnp.cumsum(df_b) - df_b + jnp.sum(df_a)
    rs_a = jnp.cumsum(sm_a) - sm_a
    rs_b = jnp.cumsum(sm_b) - sm_b + jnp.sum(sm_a)
    n_diff = jnp.sum(df_a) + jnp.sum(df_b)
    posn_a = jnp.where(same_a, n_diff + rs_a, rd_a)
    posn_b = jnp.where(same_b, n_diff + rs_b, rd_b)

    ins_a = plsc.load_gather(selrow, [posn_a])
    ins_b = plsc.load_gather(selrow, [posn_b])

    lt_a = ltrow[pl.ds(0, half)]
    lt_b = ltrow[pl.ds(half, half)]
    lb_a = lbrow[pl.ds(0, half)]
    lb_b = lbrow[pl.ds(half, half)]
    val_a = jnp.where(same_a, lb_a, lt_a)
    val_b = jnp.where(same_b, lb_b, lt_b)
    neg = jnp.float32(-1e30)
    nmax = jnp.maximum(
        jnp.max(jnp.where(ins_a > 0.5, val_a, neg)),
        jnp.max(jnp.where(ins_b > 0.5, val_b, neg)))
    sumexp = (jnp.sum(ins_a * jnp.exp(val_a - nmax))
              + jnp.sum(ins_b * jnp.exp(val_b - nmax)))

    cond_a = same_a & (ci_a != tid_i)
    cond_b = same_b & (ci_b != tid_i)
    first_a = jnp.min(jnp.where(cond_a, ci_a, 2 * TB))
    first_b = jnp.min(jnp.where(cond_b, ci_b, 2 * TB))
    firstc = jnp.minimum(first_a, first_b)
    firstc = jnp.where(firstc == 2 * TB, 0, firstc)
    pos_logit = (jnp.sum(jnp.where(ci_a == firstc, lt_a, 0.0))
                 + jnp.sum(jnp.where(ci_b == firstc, lt_b, 0.0)))

    res = (jnp.where(ci_a == 0, sumexp, 0.0)
           + jnp.where(ci_a == 1, nmax, 0.0)
           + jnp.where(ci_a == 2, pos_logit, 0.0))
    obuf[...] = res
    pltpu.sync_copy(obuf, out_hbm.at[wid])


def _final_kernel(x_ref, out_ref):
    se = x_ref[:, 0:1]
    nm = x_ref[:, 1:2]
    po = x_ref[:, 2:3]
    loss_i = jnp.log(se) - (po - nm)
    out_ref[...] = jnp.sum(loss_i, axis=0, keepdims=True) / TB


def kernel(features, labels, tasks):
    b = features.shape[0]
    task_ids = (2 * tasks[:, None]
                + jnp.arange(2, dtype=jnp.int32)[None, :]).reshape(TB)

    logits = pl.pallas_call(
        _logits_kernel,
        grid=(b,),
        in_specs=[
            pl.BlockSpec((1, D, H, W), lambda i: (i, 0, 0, 0)),
            pl.BlockSpec((1, 2, H, W), lambda i: (i, 0, 0, 0)),
        ],
        out_specs=pl.BlockSpec((2, TB, TB), lambda i: (0, 0, 0)),
        out_shape=jax.ShapeDtypeStruct((2, TB, TB), jnp.float32),
        scratch_shapes=[pltpu.VMEM((B, D, 128), jnp.float32)],
        compiler_params=pltpu.CompilerParams(
            dimension_semantics=("arbitrary",)),
    )(features, labels)

    mesh = plsc.VectorSubcoreMesh(core_axis_name="c", subcore_axis_name="s")
    sc_fn = pl.kernel(
        _sc_select_kernel,
        mesh=mesh,
        out_type=jax.ShapeDtypeStruct((TB, 16), jnp.float32),
        scratch_types=[
            pltpu.VMEM((TB,), jnp.float32),
            pltpu.VMEM((TB,), jnp.float32),
            pltpu.VMEM((TB,), jnp.float32),
            pltpu.VMEM((TB,), jnp.int32),
            pltpu.VMEM((16,), jnp.float32),
        ],
        compiler_params=pltpu.CompilerParams(needs_layout_passes=False),
    )
    terms = sc_fn(logits[0], logits[1], task_ids, jnp.asarray(_sel))

    loss = pl.pallas_call(
        _final_kernel,
        out_shape=jax.ShapeDtypeStruct((1, 1), jnp.float32),
    )(terms)
    return loss[0, 0]


# speedup vs baseline: 1.0006x; 1.0006x over previous
import numpy as np
import jax
import jax.numpy as jnp
from jax import lax
from jax.experimental import pallas as pl
from jax.experimental.pallas import tpu as pltpu
from jax.experimental.pallas import tpu_sc as plsc

N_NEG = 24
TEMP = 0.07
B, D, H, W = 16, 32, 224, 224
TB = 2 * B
HW = H * W

_perms = np.stack([np.random.default_rng(1000 + i).permutation(TB)[:N_NEG]
                   for i in range(TB)])
_sel = np.zeros((TB, TB), np.float32)
for _i in range(TB):
    _sel[_i, _perms[_i]] = 1.0


def _sc_select_kernel(tid_hbm, sel_hbm, out_hbm, selrow, tidrow, obuf):
    nc = 2
    wid = lax.axis_index("s") * nc + lax.axis_index("c")
    pltpu.sync_copy(sel_hbm.at[wid], selrow)
    pltpu.sync_copy(tid_hbm, tidrow)

    half = TB // 2
    ci_a = lax.iota(jnp.int32, 16)
    ci_b = ci_a + half
    tid_a = tidrow[pl.ds(0, half)]
    tid_b = tidrow[pl.ds(half, half)]
    tid_i = (jnp.sum(jnp.where(ci_a == wid, tid_a, 0))
             + jnp.sum(jnp.where(ci_b == wid, tid_b, 0)))

    same_a = tid_a == tid_i
    same_b = tid_b == tid_i
    df_a = jnp.where(same_a, 0, 1)
    df_b = jnp.where(same_b, 0, 1)
    sm_a = 1 - df_a
    sm_b = 1 - df_b
    rd_a = jnp.cumsum(df_a) - df_a
    rd_b = jnp.cumsum(df_b) - df_b + jnp.sum(df_a)
    rs_a = jnp.cumsum(sm_a) - sm_a
    rs_b = jnp.cumsum(sm_b) - sm_b + jnp.sum(sm_a)
    n_diff = jnp.sum(df_a) + jnp.sum(df_b)
    posn_a = jnp.where(same_a, n_diff + rs_a, rd_a)
    posn_b = jnp.where(same_b, n_diff + rs_b, rd_b)

    ins_a = plsc.load_gather(selrow, [posn_a])
    ins_b = plsc.load_gather(selrow, [posn_b])
    ins_a = jnp.where(same_a, -ins_a, ins_a)
    ins_b = jnp.where(same_b, -ins_b, ins_b)

    cond_a = same_a & (ci_a != tid_i)
    cond_b = same_b & (ci_b != tid_i)
    first_a = jnp.min(jnp.where(cond_a, ci_a, 2 * TB))
    first_b = jnp.min(jnp.where(cond_b, ci_b, 2 * TB))
    firstc = jnp.minimum(first_a, first_b)
    firstc = jnp.where(firstc == 2 * TB, 0, firstc)

    obuf[pl.ds(0, half)] = ins_a
    obuf[pl.ds(half, half)] = ins_b
    obuf[pl.ds(TB, 16)] = jnp.where(ci_a == 0, jnp.float32(firstc), 0.0)
    pltpu.sync_copy(obuf, out_hbm.at[wid])


def _fused_kernel(feat_ref, lab_ref, ins_ref, first_ref, out_ref, acc_ref):
    i = pl.program_id(0)
    f = feat_ref[0]
    l0 = lab_ref[0, 0]
    l1 = lab_ref[0, 1]
    st0 = jnp.sum(f * l0[None, :, :], axis=(1, 2), keepdims=True)
    st1 = jnp.sum(f * l1[None, :, :], axis=(1, 2), keepdims=True)
    tot = jnp.sum(f, axis=(1, 2), keepdims=True)
    acc_ref[i, :, 0:1] = st0.reshape(D, 1)
    acc_ref[i, :, 1:2] = st1.reshape(D, 1)
    acc_ref[i, :, 2:3] = tot.reshape(D, 1)
    cnt0 = jnp.sum(l0, axis=(0, 1), keepdims=True)
    cnt1 = jnp.sum(l1, axis=(0, 1), keepdims=True)
    acc_ref[i, 0:1, 3:4] = cnt0
    acc_ref[i, 0:1, 4:5] = cnt1

    @pl.when(i == B - 1)
    def _finish():
        st_t = jnp.concatenate([acc_ref[b, :, 0:2] for b in range(B)], axis=1)
        tot_t = jnp.concatenate(
            [acc_ref[b, :, 2:3] for b in range(B) for _ in range(2)], axis=1)
        cnt_r = jnp.concatenate(
            [acc_ref[b, 0:1, 3:5] for b in range(B)], axis=1)

        def normalize(v, c):
            v = v / jnp.maximum(c, 1.0)
            n = jnp.sqrt(jnp.sum(v * v, axis=0, keepdims=True))
            return v / jnp.maximum(n, 1e-12)

        tgt = normalize(st_t, cnt_r)
        bgd = normalize(tot_t - st_t, float(HW) - cnt_r)

        dn = (((0,), (0,)), ((), ()))
        lt = jax.lax.dot_general(tgt, tgt, dn,
                                 preferred_element_type=jnp.float32) / TEMP
        lb = jax.lax.dot_general(tgt, bgd, dn,
                                 preferred_element_type=jnp.float32) / TEMP

        ins = ins_ref[...]
        firstc = first_ref[...]
        sm = jnp.where(ins < -0.5, 1.0, 0.0)
        member = jnp.abs(ins)
        val = jnp.where(sm > 0.5, lb, lt)
        vmask = jnp.where(member > 0.5, val, -1e30)
        nmax = jnp.max(vmask, axis=1, keepdims=True)
        sumexp = jnp.sum(member * jnp.exp(val - nmax), axis=1, keepdims=True)

        cc = jax.lax.broadcasted_iota(jnp.int32, (TB, TB), 1)
        pos_logit = jnp.sum(
            lt * jnp.where(cc == firstc.astype(jnp.int32), 1.0, 0.0),
            axis=1, keepdims=True)

        loss_i = jnp.log(sumexp) - (pos_logit - nmax)
        out_ref[...] = jnp.sum(loss_i, axis=0, keepdims=True) / TB


def kernel(features, labels, tasks):
    b = features.shape[0]
    task_ids = (2 * tasks[:, None]
                + jnp.arange(2, dtype=jnp.int32)[None, :]).reshape(TB)

    mesh = plsc.VectorSubcoreMesh(core_axis_name="c", subcore_axis_name="s")
    sc_fn = pl.kernel(
        _sc_select_kernel,
        mesh=mesh,
        out_type=jax.ShapeDtypeStruct((TB, TB + 16), jnp.float32),
        scratch_types=[
            pltpu.VMEM((TB,), jnp.float32),
            pltpu.VMEM((TB,), jnp.int32),
            pltpu.VMEM((TB + 16,), jnp.float32),
        ],
        compiler_params=pltpu.CompilerParams(needs_layout_passes=False),
    )
    select = sc_fn(task_ids, jnp.asarray(_sel))
    ins = select[:, 0:TB]
    firstc = select[:, TB:TB + 1]

    loss = pl.pallas_call(
        _fused_kernel,
        grid=(b,),
        in_specs=[
            pl.BlockSpec((1, D, H, W), lambda i: (i, 0, 0, 0)),
            pl.BlockSpec((1, 2, H, W), lambda i: (i, 0, 0, 0)),
            pl.BlockSpec((TB, TB), lambda i: (0, 0)),
            pl.BlockSpec((TB, 1), lambda i: (0, 0)),
        ],
        out_specs=pl.BlockSpec((1, 1), lambda i: (0, 0)),
        out_shape=jax.ShapeDtypeStruct((1, 1), jnp.float32),
        scratch_shapes=[pltpu.VMEM((B, D, 128), jnp.float32)],
        compiler_params=pltpu.CompilerParams(
            dimension_semantics=("arbitrary",)),
    )(features, labels, ins, firstc)
    return loss[0, 0]
